# Initial kernel scaffold; baseline (speedup 1.0000x reference)
import jax
import jax.numpy as jnp
from jax.experimental import pallas as pl


def kernel(x, edge_index, w1, b1, w2, b2, g1, be1, w3, b3, w4, b4, g2, be2):
    raise NotImplementedError("write your pallas kernel here")


# speedup vs baseline: 1.7535x; 1.7535x over previous
import functools

import jax
import jax.numpy as jnp
from jax.experimental import pallas as pl
from jax.experimental.pallas import tpu as pltpu

BN_EPS = 1e-5


def _round_kernel(x_ref, o_ref):
    o_ref[...] = x_ref[...].astype(jnp.bfloat16).astype(jnp.float32)


def _round_bf16(x, *, n, w, tm):
    return pl.pallas_call(
        _round_kernel,
        out_shape=jax.ShapeDtypeStruct((n, w), jnp.float32),
        grid=(n // tm,),
        in_specs=[pl.BlockSpec((tm, w), lambda t: (t, 0))],
        out_specs=pl.BlockSpec((tm, w), lambda t: (t, 0)),
        compiler_params=pltpu.CompilerParams(
            dimension_semantics=("parallel",),
            vmem_limit_bytes=32 * 2**20,
        ),
    )(x)


def _agg_kernel(src_ref, dst_ref, x_ref, out_ref, smem_s, smem_d,
                sem_s, sem_d, *, chunk, unroll):
    k = pl.program_id(1)

    @pl.when(k == 0)
    def _():
        out_ref[...] = jnp.zeros_like(out_ref)

    cp_s = pltpu.make_async_copy(src_ref.at[0, 0], smem_s, sem_s)
    cp_d = pltpu.make_async_copy(dst_ref.at[0, 0], smem_d, sem_d)
    cp_s.start()
    cp_d.start()
    cp_s.wait()
    cp_d.wait()

    def body(c, carry):
        base = c * unroll
        for u in range(unroll):
            s = smem_s[base + u]
            d = smem_d[base + u]
            out_ref[d, 0] = out_ref[d, 0] + x_ref[s, 0]
        return carry

    jax.lax.fori_loop(0, chunk // unroll, body, 0)


def _aggregate(x3, src_c, dst_c, *, n, f, chunk, nc, nk, unroll):
    return pl.pallas_call(
        functools.partial(_agg_kernel, chunk=chunk, unroll=unroll),
        out_shape=jax.ShapeDtypeStruct((nc * n, 1, f), jnp.float32),
        grid=(nc, nk),
        in_specs=[
            pl.BlockSpec((1, 1, chunk), lambda i, k: (i * nk + k, 0, 0)),
            pl.BlockSpec((1, 1, chunk), lambda i, k: (i * nk + k, 0, 0)),
            pl.BlockSpec((n, 1, f), lambda i, k: (0, 0, 0)),
        ],
        out_specs=pl.BlockSpec((n, 1, f), lambda i, k: (i, 0, 0)),
        scratch_shapes=[
            pltpu.SMEM((chunk,), jnp.int32),
            pltpu.SMEM((chunk,), jnp.int32),
            pltpu.SemaphoreType.DMA,
            pltpu.SemaphoreType.DMA,
        ],
        compiler_params=pltpu.CompilerParams(
            dimension_semantics=("parallel", "arbitrary"),
            vmem_limit_bytes=56 * 2**20,
        ),
    )(src_c, dst_c, x3)


def _mlp_kernel(p_ref, xs_ref, wa_ref, ba_ref, wb_ref, bb_ref, h_ref, st_ref):
    m = p_ref[0] + p_ref[1] + xs_ref[...]
    h = jnp.maximum(jnp.dot(m, wa_ref[...], preferred_element_type=jnp.float32)
                    + ba_ref[...], 0.0)
    h = jnp.maximum(jnp.dot(h, wb_ref[...], preferred_element_type=jnp.float32)
                    + bb_ref[...], 0.0)
    h_ref[...] = h.astype(h_ref.dtype)
    st_ref[0:1, :] = jnp.sum(h, axis=0, keepdims=True)
    st_ref[1:2, :] = jnp.sum(h * h, axis=0, keepdims=True)


def _mlp(parts, xs, wa, ba, wb, bb, *, n, f, hd, w, tm):
    nt = n // tm
    return pl.pallas_call(
        _mlp_kernel,
        out_shape=(jax.ShapeDtypeStruct((n, w), jnp.bfloat16),
                   jax.ShapeDtypeStruct((nt * 8, w), jnp.float32)),
        grid=(nt,),
        in_specs=[
            pl.BlockSpec((2, tm, f), lambda t: (0, t, 0)),
            pl.BlockSpec((tm, f), lambda t: (t, 0)),
            pl.BlockSpec((f, hd), lambda t: (0, 0)),
            pl.BlockSpec((1, hd), lambda t: (0, 0)),
            pl.BlockSpec((hd, w), lambda t: (0, 0)),
            pl.BlockSpec((1, w), lambda t: (0, 0)),
        ],
        out_specs=(
            pl.BlockSpec((tm, w), lambda t: (t, 0)),
            pl.BlockSpec((8, w), lambda t: (t, 0)),
        ),
        compiler_params=pltpu.CompilerParams(
            dimension_semantics=("parallel",),
            vmem_limit_bytes=48 * 2**20,
        ),
    )(parts, xs, wa, ba, wb, bb)


def _finish_bn(stats, g, be, n):
    wdt = stats.shape[-1]
    st = stats.reshape(-1, 8, wdt)
    total = jnp.sum(st[:, 0, :], axis=0)
    totsq = jnp.sum(st[:, 1, :], axis=0)
    mean = total / n
    var = jnp.maximum(totsq / n - mean * mean, 0.0)
    scale = g * jax.lax.rsqrt(var + BN_EPS)
    shift = be - mean * scale
    return jnp.zeros((8, wdt), jnp.float32).at[0].set(scale).at[1].set(shift)


def _bn_kernel(h_ref, ss_ref, o_ref):
    o_ref[...] = h_ref[...].astype(jnp.float32) * ss_ref[0:1, :] + ss_ref[1:2, :]


def _bn_apply(h, ss, *, n, w, tm):
    nt = n // tm
    return pl.pallas_call(
        _bn_kernel,
        out_shape=jax.ShapeDtypeStruct((n, w), jnp.float32),
        grid=(nt,),
        in_specs=[pl.BlockSpec((tm, w), lambda t: (t, 0)),
                  pl.BlockSpec((8, w), lambda t: (0, 0))],
        out_specs=pl.BlockSpec((tm, w), lambda t: (t, 0)),
        compiler_params=pltpu.CompilerParams(
            dimension_semantics=("parallel",),
            vmem_limit_bytes=32 * 2**20,
        ),
    )(h, ss)


def kernel(x, edge_index, w1, b1, w2, b2, g1, be1, w3, b3, w4, b4, g2, be2):
    n, f = x.shape
    dim = w1.shape[1]
    out_dim = w4.shape[1]
    e = edge_index.shape[1]
    out_p = 128
    nc = 2
    chunk = min(8192, e // nc)
    nk = e // (nc * chunk)
    tm = min(512, n)
    unroll = 8

    src = edge_index[0].reshape(nc * nk, 1, chunk)
    dst = edge_index[1].reshape(nc * nk, 1, chunk)

    xr = _round_bf16(x, n=n, w=f, tm=tm)
    parts1 = _aggregate(xr.reshape(n, 1, f), src, dst, n=n, f=f, chunk=chunk,
                        nc=nc, nk=nk, unroll=unroll)
    h1, st1 = _mlp(parts1.reshape(nc, n, f), x, w1, b1, w2, b2,
                   n=n, f=f, hd=dim, w=dim, tm=tm)
    ss1 = _finish_bn(st1, g1.reshape(-1), be1.reshape(-1), n)
    hid = _bn_apply(h1, ss1, n=n, w=dim, tm=tm)

    hidr = _round_bf16(hid, n=n, w=dim, tm=tm)
    parts2 = _aggregate(hidr.reshape(n, 1, dim), src, dst, n=n, f=dim,
                        chunk=chunk, nc=nc, nk=nk, unroll=unroll)
    w4p = jnp.zeros((dim, out_p), jnp.float32).at[:, :out_dim].set(w4)
    b4p = jnp.zeros((1, out_p), jnp.float32).at[:, :out_dim].set(b4)
    h2, st2 = _mlp(parts2.reshape(nc, n, dim), hid, w3, b3, w4p, b4p,
                   n=n, f=dim, hd=dim, w=out_p, tm=tm)
    g2p = jnp.zeros((out_p,), jnp.float32).at[:out_dim].set(g2.reshape(-1))
    be2p = jnp.zeros((out_p,), jnp.float32).at[:out_dim].set(be2.reshape(-1))
    ss2 = _finish_bn(st2, g2p, be2p, n)
    out = _bn_apply(h2, ss2, n=n, w=out_p, tm=tm)

    return out[:, :out_dim], hid
